# baseline (speedup 1.0000x reference)
import functools

import jax
import jax.numpy as jnp
from jax import lax
from jax.experimental import pallas as pl
from jax.experimental.pallas import tpu as pltpu
from jax.experimental.pallas import tpu_sc as plsc

BATCH = 16384
FIELDS = 26
EMBED_DIM = 16

NC = 1
NS = 16
NW = NC * NS

BBLK = BATCH // NW
LANES = 16
NPAIR = FIELDS // 2


@functools.partial(
    pl.kernel,
    out_type=jax.ShapeDtypeStruct((FIELDS, EMBED_DIM, BATCH), jnp.float32),
    mesh=plsc.VectorSubcoreMesh(core_axis_name="c", subcore_axis_name="s", num_cores=1),
    scratch_types=[
        pltpu.VMEM((FIELDS, BBLK), jnp.int32),
        pltpu.VMEM((BBLK, EMBED_DIM), jnp.float32),
        pltpu.VMEM((BBLK, EMBED_DIM), jnp.float32),
        pltpu.VMEM((EMBED_DIM, BBLK), jnp.float32),
        pltpu.VMEM((EMBED_DIM, BBLK), jnp.float32),
        pltpu.SemaphoreType.DMA,
        pltpu.SemaphoreType.DMA,
        pltpu.SemaphoreType.DMA,
    ],
    compiler_params=pltpu.CompilerParams(
        use_tc_tiling_on_sc=False, needs_layout_passes=False
    ),
)
def _embed_kernel(xt_hbm, tab_hbm, out_hbm,
                  idx_v, rows_a, rows_b, t_a, t_b, sem_a, sem_b, sem_o):
    wid = lax.axis_index("s") * NC + lax.axis_index("c")
    b0 = wid * BBLK
    pltpu.sync_copy(xt_hbm.at[:, pl.ds(b0, BBLK)], idx_v)

    lanes = lax.iota(jnp.int32, LANES)

    def extract(rows_v, t_v):
        def block(c, carry):
            row = lanes + c * LANES
            for k in range(EMBED_DIM):
                col = (lanes + k) & 15
                vals = plsc.load_gather(rows_v, [row, col])
                plsc.store_scatter(t_v, [col, row], vals)
            return carry

        lax.fori_loop(0, BBLK // LANES, block, 0)

    ga0 = pltpu.async_copy(tab_hbm.at[idx_v.at[0]], rows_a, sem_a)

    def pair_body(i, carry):
        f0 = 2 * i
        pltpu.make_async_copy(tab_hbm.at[idx_v.at[f0]], rows_a, sem_a).wait()
        pltpu.async_copy(tab_hbm.at[idx_v.at[f0 + 1]], rows_b, sem_b)
        extract(rows_a, t_a)
        pltpu.async_copy(t_a, out_hbm.at[f0, :, pl.ds(b0, BBLK)], sem_o).wait()
        pltpu.make_async_copy(
            tab_hbm.at[idx_v.at[f0 + 1]], rows_b, sem_b).wait()

        @pl.when(i < NPAIR - 1)
        def _():
            pltpu.async_copy(tab_hbm.at[idx_v.at[f0 + 2]], rows_a, sem_a)

        extract(rows_b, t_b)
        pltpu.async_copy(
            t_b, out_hbm.at[f0 + 1, :, pl.ds(b0, BBLK)], sem_o).wait()
        return carry

    lax.fori_loop(0, NPAIR, pair_body, 0)


def kernel(x, mask, table):
    del mask
    outt = _embed_kernel(x.T, table)
    return outt.transpose(2, 0, 1)


# speedup vs baseline: 1.1022x; 1.1022x over previous
import functools

import jax
import jax.numpy as jnp
from jax import lax
from jax.experimental import pallas as pl
from jax.experimental.pallas import tpu as pltpu
from jax.experimental.pallas import tpu_sc as plsc

BATCH = 16384
FIELDS = 26
EMBED_DIM = 16

NC = 2
NS = 16
NW = NC * NS

BBLK = BATCH // NW
LANES = 16
NPAIR = FIELDS // 2


@functools.partial(
    pl.kernel,
    out_type=jax.ShapeDtypeStruct((FIELDS, EMBED_DIM, BATCH), jnp.float32),
    mesh=plsc.VectorSubcoreMesh(core_axis_name="c", subcore_axis_name="s"),
    scratch_types=[
        pltpu.VMEM((FIELDS * BBLK,), jnp.int32),
        pltpu.VMEM((FIELDS, BBLK), jnp.int32),
        pltpu.VMEM((BBLK, EMBED_DIM), jnp.float32),
        pltpu.VMEM((BBLK, EMBED_DIM), jnp.float32),
        pltpu.VMEM((EMBED_DIM, BBLK), jnp.float32),
        pltpu.VMEM((EMBED_DIM, BBLK), jnp.float32),
        pltpu.SemaphoreType.DMA,
        pltpu.SemaphoreType.DMA,
        pltpu.SemaphoreType.DMA,
    ],
    compiler_params=pltpu.CompilerParams(
        use_tc_tiling_on_sc=False, needs_layout_passes=False
    ),
)
def _embed_kernel(xf_hbm, tab_hbm, out_hbm,
                  raw_v, idx_v, rows_a, rows_b, t_a, t_b, sem_a, sem_b, sem_o):
    wid = lax.axis_index("s") * NC + lax.axis_index("c")
    b0 = wid * BBLK
    pltpu.sync_copy(xf_hbm.at[pl.ds(b0 * FIELDS, BBLK * FIELDS)], raw_v)

    lanes = lax.iota(jnp.int32, LANES)

    def deint(c, carry):
        base = (lanes + c * LANES) * FIELDS
        for f in range(FIELDS):
            idx_v[f, pl.ds(c * LANES, LANES)] = plsc.load_gather(
                raw_v, [base + f])
        return carry

    lax.fori_loop(0, BBLK // LANES, deint, 0)

    def extract(rows_v, t_v):
        def block(c, carry):
            row = lanes + c * LANES
            for k in range(EMBED_DIM):
                col = (lanes + k) & 15
                vals = plsc.load_gather(rows_v, [row, col])
                plsc.store_scatter(t_v, [col, row], vals)
            return carry

        lax.fori_loop(0, BBLK // LANES, block, 0)

    pltpu.async_copy(tab_hbm.at[idx_v.at[0]], rows_a, sem_a)

    def pair_body(i, carry):
        f0 = 2 * i
        pltpu.make_async_copy(tab_hbm.at[idx_v.at[f0]], rows_a, sem_a).wait()
        pltpu.async_copy(tab_hbm.at[idx_v.at[f0 + 1]], rows_b, sem_b)
        extract(rows_a, t_a)
        pltpu.async_copy(t_a, out_hbm.at[f0, :, pl.ds(b0, BBLK)], sem_o).wait()
        pltpu.make_async_copy(
            tab_hbm.at[idx_v.at[f0 + 1]], rows_b, sem_b).wait()

        @pl.when(i < NPAIR - 1)
        def _():
            pltpu.async_copy(tab_hbm.at[idx_v.at[f0 + 2]], rows_a, sem_a)

        extract(rows_b, t_b)
        pltpu.async_copy(
            t_b, out_hbm.at[f0 + 1, :, pl.ds(b0, BBLK)], sem_o).wait()
        return carry

    lax.fori_loop(0, NPAIR, pair_body, 0)


def kernel(x, mask, table):
    del mask
    outt = _embed_kernel(x.reshape(-1), table)
    return outt.transpose(2, 0, 1)


# speedup vs baseline: 1.5123x; 1.3720x over previous
import functools

import jax
import jax.numpy as jnp
from jax import lax
from jax.experimental import pallas as pl
from jax.experimental.pallas import tpu as pltpu
from jax.experimental.pallas import tpu_sc as plsc

BATCH = 16384
FIELDS = 26
EMBED_DIM = 16
VOCAB = 1000000

NC = 2
NS = 16
NW = NC * NS

BBLK = BATCH // NW
LANES = 16
NPAIR = FIELDS // 2

GROUPS = VOCAB // 8
TCOLS = VOCAB // 128
TC_PER_W = TCOLS // NW
TC_REM = TCOLS - TC_PER_W * NW
NPAIR_TC = TC_PER_W // 2


@functools.partial(
    pl.kernel,
    out_type=jax.ShapeDtypeStruct((GROUPS, 128), jnp.float32),
    mesh=plsc.VectorSubcoreMesh(core_axis_name="c", subcore_axis_name="s"),
    scratch_types=[
        pltpu.VMEM((EMBED_DIM, 128), jnp.float32),
        pltpu.VMEM((EMBED_DIM, 128), jnp.float32),
        pltpu.VMEM((EMBED_DIM, 128), jnp.float32),
        pltpu.VMEM((EMBED_DIM, 128), jnp.float32),
        pltpu.VMEM((EMBED_DIM, 64), jnp.float32),
        pltpu.SemaphoreType.DMA,
        pltpu.SemaphoreType.DMA,
        pltpu.SemaphoreType.DMA,
    ],
    compiler_params=pltpu.CompilerParams(
        use_tc_tiling_on_sc=True, needs_layout_passes=False
    ),
)
def _convert_kernel(tabt_hbm, out_hbm, s_a, s_b, d_a, d_b, s_t,
                    sem_a, sem_b, sem_o):
    wid = lax.axis_index("s") * NC + lax.axis_index("c")
    lanes = lax.iota(jnp.int32, LANES)

    dvs = [(lanes + t) & 15 for t in range(16)]
    cds = [(lanes & 7) * EMBED_DIM + dv for dv in dvs]

    def shuffle(s_v, d_v):
        for c in range(8):
            col_src = lanes + 16 * c
            qv = (lanes >> 3) + 2 * c
            for t in range(16):
                vals = plsc.load_gather(s_v, [dvs[t], col_src])
                plsc.store_scatter(d_v, [qv, cds[t]], vals)

    def col_in(tc, s_v, sem):
        return pltpu.async_copy(tabt_hbm.at[:, pl.ds(tc * 128, 128)], s_v, sem)

    def col_out(tc, d_v):
        pltpu.async_copy(d_v, out_hbm.at[pl.ds(tc * 16, 16), :], sem_o).wait()

    tc0 = wid * TC_PER_W
    col_in(tc0, s_a, sem_a)

    def pair_body(i, carry):
        tc = tc0 + 2 * i
        pltpu.make_async_copy(
            tabt_hbm.at[:, pl.ds(tc * 128, 128)], s_a, sem_a).wait()
        col_in(tc + 1, s_b, sem_b)
        shuffle(s_a, d_a)
        col_out(tc, d_a)
        pltpu.make_async_copy(
            tabt_hbm.at[:, pl.ds(tc * 128, 128)], s_b, sem_b).wait()

        @pl.when(i < NPAIR_TC - 1)
        def _():
            col_in(tc + 2, s_a, sem_a)

        shuffle(s_b, d_b)
        col_out(tc + 1, d_b)
        return carry

    lax.fori_loop(0, NPAIR_TC, pair_body, 0)

    @pl.when(wid < TC_REM)
    def _():
        tc = NW * TC_PER_W + wid
        col_in(tc, s_a, sem_a).wait()
        shuffle(s_a, d_a)
        col_out(tc, d_a)

    @pl.when(wid == TC_REM)
    def _():
        r0 = TCOLS * 128
        pltpu.async_copy(tabt_hbm.at[:, pl.ds(r0, 64)], s_t, sem_a).wait()
        for c in range(4):
            col_src = lanes + 16 * c
            qv = (lanes >> 3) + 2 * c
            for t in range(16):
                vals = plsc.load_gather(s_t, [dvs[t], col_src])
                plsc.store_scatter(d_a, [qv, cds[t]], vals)
        pltpu.async_copy(
            d_a.at[pl.ds(0, 8), :], out_hbm.at[pl.ds(TCOLS * 16, 8), :], sem_o
        ).wait()


@functools.partial(
    pl.kernel,
    out_type=jax.ShapeDtypeStruct((FIELDS, EMBED_DIM, BATCH), jnp.float32),
    mesh=plsc.VectorSubcoreMesh(core_axis_name="c", subcore_axis_name="s"),
    scratch_types=[
        pltpu.VMEM((FIELDS * BBLK,), jnp.int32),
        pltpu.VMEM((FIELDS, BBLK), jnp.int32),
        pltpu.VMEM((BBLK, EMBED_DIM), jnp.float32),
        pltpu.VMEM((BBLK, EMBED_DIM), jnp.float32),
        pltpu.VMEM((EMBED_DIM, BBLK), jnp.float32),
        pltpu.VMEM((EMBED_DIM, BBLK), jnp.float32),
        pltpu.SemaphoreType.DMA,
        pltpu.SemaphoreType.DMA,
        pltpu.SemaphoreType.DMA,
    ],
    compiler_params=pltpu.CompilerParams(
        use_tc_tiling_on_sc=False, needs_layout_passes=False
    ),
)
def _embed_kernel(xf_hbm, tab_hbm, out_hbm,
                  raw_v, idx_v, rows_a, rows_b, t_a, t_b, sem_a, sem_b, sem_o):
    wid = lax.axis_index("s") * NC + lax.axis_index("c")
    b0 = wid * BBLK
    pltpu.sync_copy(xf_hbm.at[pl.ds(b0 * FIELDS, BBLK * FIELDS)], raw_v)

    lanes = lax.iota(jnp.int32, LANES)

    def deint(c, carry):
        base = (lanes + c * LANES) * FIELDS
        for f in range(FIELDS):
            idx_v[f, pl.ds(c * LANES, LANES)] = plsc.load_gather(
                raw_v, [base + f])
        return carry

    lax.fori_loop(0, BBLK // LANES, deint, 0)

    def extract(rows_v, t_v):
        def block(c, carry):
            row = lanes + c * LANES
            for k in range(EMBED_DIM):
                col = (lanes + k) & 15
                vals = plsc.load_gather(rows_v, [row, col])
                plsc.store_scatter(t_v, [col, row], vals)
            return carry

        lax.fori_loop(0, BBLK // LANES, block, 0)

    pltpu.async_copy(tab_hbm.at[idx_v.at[0]], rows_a, sem_a)

    def pair_body(i, carry):
        f0 = 2 * i
        pltpu.make_async_copy(tab_hbm.at[idx_v.at[f0]], rows_a, sem_a).wait()
        pltpu.async_copy(tab_hbm.at[idx_v.at[f0 + 1]], rows_b, sem_b)
        extract(rows_a, t_a)
        pltpu.async_copy(t_a, out_hbm.at[f0, :, pl.ds(b0, BBLK)], sem_o).wait()
        pltpu.make_async_copy(
            tab_hbm.at[idx_v.at[f0 + 1]], rows_b, sem_b).wait()

        @pl.when(i < NPAIR - 1)
        def _():
            pltpu.async_copy(tab_hbm.at[idx_v.at[f0 + 2]], rows_a, sem_a)

        extract(rows_b, t_b)
        pltpu.async_copy(
            t_b, out_hbm.at[f0 + 1, :, pl.ds(b0, BBLK)], sem_o).wait()
        return carry

    lax.fori_loop(0, NPAIR, pair_body, 0)


def kernel(x, mask, table):
    del mask
    grouped = _convert_kernel(table.T)
    outt = _embed_kernel(x.reshape(-1), grouped.reshape(VOCAB, EMBED_DIM))
    return outt.transpose(2, 0, 1)


# speedup vs baseline: 1.8961x; 1.2538x over previous
import functools

import jax
import jax.numpy as jnp
from jax import lax
from jax.experimental import pallas as pl
from jax.experimental.pallas import tpu as pltpu
from jax.experimental.pallas import tpu_sc as plsc

BATCH = 16384
FIELDS = 26
EMBED_DIM = 16
VOCAB = 1000000

NC = 2
NS = 16
NW = NC * NS

BBLK = BATCH // NW
LANES = 16
NPAIR = FIELDS // 2

GROUPS = VOCAB // 8
TCOLS = VOCAB // 128
TC_PER_W = TCOLS // NW
TC_REM = TCOLS - TC_PER_W * NW
CBLK = 4
NBLK_TC = TC_PER_W // CBLK
NPAIR_TC = NBLK_TC // 2


@functools.partial(
    pl.kernel,
    out_type=jax.ShapeDtypeStruct((GROUPS, 128), jnp.float32),
    mesh=plsc.VectorSubcoreMesh(core_axis_name="c", subcore_axis_name="s"),
    scratch_types=[
        pltpu.VMEM((EMBED_DIM, CBLK * 128), jnp.float32),
        pltpu.VMEM((EMBED_DIM, CBLK * 128), jnp.float32),
        pltpu.VMEM((CBLK * 16, 128), jnp.float32),
        pltpu.VMEM((CBLK * 16, 128), jnp.float32),
        pltpu.VMEM((EMBED_DIM, 64), jnp.float32),
        pltpu.SemaphoreType.DMA,
        pltpu.SemaphoreType.DMA,
        pltpu.SemaphoreType.DMA,
    ],
    compiler_params=pltpu.CompilerParams(
        use_tc_tiling_on_sc=True, needs_layout_passes=False
    ),
)
def _convert_kernel(tabt_hbm, out_hbm, s_a, s_b, d_a, d_b, s_t,
                    sem_a, sem_b, sem_o):
    wid = lax.axis_index("s") * NC + lax.axis_index("c")
    lanes = lax.iota(jnp.int32, LANES)

    dvs = [(lanes + t) & 15 for t in range(16)]
    cds = [(lanes & 7) * EMBED_DIM + dv for dv in dvs]

    def shuffle(s_v, d_v):
        def one_col(cc, carry):
            for c in range(8):
                col_src = lanes + (16 * c) + 128 * cc
                qv = (lanes >> 3) + (2 * c) + 16 * cc
                for t in range(16):
                    vals = plsc.load_gather(s_v, [dvs[t], col_src])
                    plsc.store_scatter(d_v, [qv, cds[t]], vals)
            return carry

        lax.fori_loop(0, CBLK, one_col, 0)

    def blk_in(tc, s_v, sem):
        return pltpu.async_copy(
            tabt_hbm.at[:, pl.ds(tc * 128, CBLK * 128)], s_v, sem)

    def blk_out(tc, d_v):
        pltpu.async_copy(
            d_v, out_hbm.at[pl.ds(tc * 16, CBLK * 16), :], sem_o).wait()

    tc0 = wid * TC_PER_W
    blk_in(tc0, s_a, sem_a)

    def pair_body(i, carry):
        tc = tc0 + 2 * CBLK * i
        pltpu.make_async_copy(
            tabt_hbm.at[:, pl.ds(tc * 128, CBLK * 128)], s_a, sem_a).wait()
        blk_in(tc + CBLK, s_b, sem_b)
        shuffle(s_a, d_a)
        blk_out(tc, d_a)
        pltpu.make_async_copy(
            tabt_hbm.at[:, pl.ds(tc * 128, CBLK * 128)], s_b, sem_b).wait()
        blk_in(tc + 2 * CBLK, s_a, sem_a)
        shuffle(s_b, d_b)
        blk_out(tc + CBLK, d_b)
        return carry

    lax.fori_loop(0, NPAIR_TC, pair_body, 0)

    tc_last = tc0 + 2 * CBLK * NPAIR_TC
    pltpu.make_async_copy(
        tabt_hbm.at[:, pl.ds(tc_last * 128, CBLK * 128)], s_a, sem_a).wait()
    shuffle(s_a, d_a)
    blk_out(tc_last, d_a)

    @pl.when(wid < TC_REM)
    def _():
        tc = NW * TC_PER_W + wid
        pltpu.async_copy(
            tabt_hbm.at[:, pl.ds(tc * 128, 128)], s_b.at[:, pl.ds(0, 128)],
            sem_b).wait()
        for c in range(8):
            col_src = lanes + 16 * c
            qv = (lanes >> 3) + 2 * c
            for t in range(16):
                vals = plsc.load_gather(s_b, [dvs[t], col_src])
                plsc.store_scatter(d_b, [qv, cds[t]], vals)
        pltpu.async_copy(
            d_b.at[pl.ds(0, 16), :], out_hbm.at[pl.ds(tc * 16, 16), :],
            sem_o).wait()

    @pl.when(wid == TC_REM)
    def _():
        r0 = TCOLS * 128
        pltpu.async_copy(tabt_hbm.at[:, pl.ds(r0, 64)], s_t, sem_a).wait()
        for c in range(4):
            col_src = lanes + 16 * c
            qv = (lanes >> 3) + 2 * c
            for t in range(16):
                vals = plsc.load_gather(s_t, [dvs[t], col_src])
                plsc.store_scatter(d_a, [qv, cds[t]], vals)
        pltpu.async_copy(
            d_a.at[pl.ds(0, 8), :], out_hbm.at[pl.ds(TCOLS * 16, 8), :], sem_o
        ).wait()


@functools.partial(
    pl.kernel,
    out_type=jax.ShapeDtypeStruct((FIELDS, EMBED_DIM, BATCH), jnp.float32),
    mesh=plsc.VectorSubcoreMesh(core_axis_name="c", subcore_axis_name="s"),
    scratch_types=[
        pltpu.VMEM((FIELDS * BBLK,), jnp.int32),
        pltpu.VMEM((FIELDS, BBLK), jnp.int32),
        pltpu.VMEM((BBLK, EMBED_DIM), jnp.float32),
        pltpu.VMEM((BBLK, EMBED_DIM), jnp.float32),
        pltpu.VMEM((EMBED_DIM, BBLK), jnp.float32),
        pltpu.VMEM((EMBED_DIM, BBLK), jnp.float32),
        pltpu.SemaphoreType.DMA,
        pltpu.SemaphoreType.DMA,
        pltpu.SemaphoreType.DMA,
    ],
    compiler_params=pltpu.CompilerParams(
        use_tc_tiling_on_sc=False, needs_layout_passes=False
    ),
)
def _embed_kernel(xf_hbm, tab_hbm, out_hbm,
                  raw_v, idx_v, rows_a, rows_b, t_a, t_b, sem_a, sem_b, sem_o):
    wid = lax.axis_index("s") * NC + lax.axis_index("c")
    b0 = wid * BBLK
    pltpu.sync_copy(xf_hbm.at[pl.ds(b0 * FIELDS, BBLK * FIELDS)], raw_v)

    lanes = lax.iota(jnp.int32, LANES)

    def deint(c, carry):
        base = (lanes + c * LANES) * FIELDS
        for f in range(FIELDS):
            idx_v[f, pl.ds(c * LANES, LANES)] = plsc.load_gather(
                raw_v, [base + f])
        return carry

    lax.fori_loop(0, BBLK // LANES, deint, 0)

    def extract(rows_v, t_v):
        def block(c, carry):
            row = lanes + c * LANES
            for k in range(EMBED_DIM):
                col = (lanes + k) & 15
                vals = plsc.load_gather(rows_v, [row, col])
                plsc.store_scatter(t_v, [col, row], vals)
            return carry

        lax.fori_loop(0, BBLK // LANES, block, 0)

    pltpu.async_copy(tab_hbm.at[idx_v.at[0]], rows_a, sem_a)

    def pair_body(i, carry):
        f0 = 2 * i
        pltpu.make_async_copy(tab_hbm.at[idx_v.at[f0]], rows_a, sem_a).wait()
        pltpu.async_copy(tab_hbm.at[idx_v.at[f0 + 1]], rows_b, sem_b)
        extract(rows_a, t_a)
        pltpu.async_copy(t_a, out_hbm.at[f0, :, pl.ds(b0, BBLK)], sem_o).wait()
        pltpu.make_async_copy(
            tab_hbm.at[idx_v.at[f0 + 1]], rows_b, sem_b).wait()

        @pl.when(i < NPAIR - 1)
        def _():
            pltpu.async_copy(tab_hbm.at[idx_v.at[f0 + 2]], rows_a, sem_a)

        extract(rows_b, t_b)
        pltpu.async_copy(
            t_b, out_hbm.at[f0 + 1, :, pl.ds(b0, BBLK)], sem_o).wait()
        return carry

    lax.fori_loop(0, NPAIR, pair_body, 0)


def kernel(x, mask, table):
    del mask
    grouped = _convert_kernel(table.T)
    outt = _embed_kernel(x.reshape(-1), grouped.reshape(VOCAB, EMBED_DIM))
    return outt.transpose(2, 0, 1)
